# baseline (speedup 1.0000x reference)
import functools

import jax
import jax.numpy as jnp
from jax import lax
from jax.experimental import pallas as pl
from jax.experimental.pallas import tpu as pltpu
from jax.experimental.pallas import tpu_sc as plsc

B, N, K_KNN, OUT_K, OUT_DIM = 16, 2048, 20, 16, 128
NPOINT = 512
EPS = 1e-5
NEG = -3.0e38

RB = 256
RQ = 256


def _knn_kernel(xr_ref, xc_ref, wa_ref, wq_ref, bq_ref, idx_ref, p_ref, q_ref):
    b = pl.program_id(0)
    xr = xr_ref[0]
    xc = xc_ref[0]
    p_ref[0] = jnp.dot(xr, wa_ref[...], preferred_element_type=jnp.float32)
    q_ref[0] = (jnp.dot(xr, wq_ref[...], preferred_element_type=jnp.float32)
                + bq_ref[...])
    xx_r = jnp.sum(xr * xr, axis=1, keepdims=True)
    xx_c = jnp.sum(xc * xc, axis=0, keepdims=True)
    pd = (2.0 * jnp.dot(xr, xc, preferred_element_type=jnp.float32)
          - xx_r - xx_c)
    iota = jax.lax.broadcasted_iota(jnp.int32, (RB, N), 1)
    base = b * N
    for t in range(K_KNN):
        m = jnp.max(pd, axis=1, keepdims=True)
        sel = pd >= m
        idxf = jnp.sum(jnp.where(sel, iota, 0), axis=1, keepdims=True)
        idxv = jnp.minimum(idxf, N - 1)
        idx_ref[0, :, pl.ds(t, 1)] = idxv + base
        pd = jnp.where(sel, NEG, pd)


def _knn_topk(x, xt, wa, wq, bq):
    return pl.pallas_call(
        _knn_kernel,
        grid=(B, N // RB),
        in_specs=[
            pl.BlockSpec((1, RB, 3), lambda b, r: (b, r, 0)),
            pl.BlockSpec((1, 3, N), lambda b, r: (b, 0, 0)),
            pl.BlockSpec((3, 64), lambda b, r: (0, 0)),
            pl.BlockSpec((3, 64), lambda b, r: (0, 0)),
            pl.BlockSpec((1, 64), lambda b, r: (0, 0)),
        ],
        out_specs=[
            pl.BlockSpec((1, RB, K_KNN), lambda b, r: (b, r, 0)),
            pl.BlockSpec((1, RB, 64), lambda b, r: (b, r, 0)),
            pl.BlockSpec((1, RB, 64), lambda b, r: (b, r, 0)),
        ],
        out_shape=[
            jax.ShapeDtypeStruct((B, N, K_KNN), jnp.int32),
            jax.ShapeDtypeStruct((B, N, 64), jnp.float32),
            jax.ShapeDtypeStruct((B, N, 64), jnp.float32),
        ],
    )(x, xt, wa, wq, bq)


def _fps_kernel(xt_ref, newp_ref, dist_ref):
    dist_ref[...] = jnp.full((B, N), 1e10, jnp.float32)
    iota = jax.lax.broadcasted_iota(jnp.int32, (B, N), 1)
    iota_out = jax.lax.broadcasted_iota(jnp.int32, (B, NPOINT), 1)

    def body(i, far):
        px = xt_ref[0]
        py = xt_ref[1]
        pz = xt_ref[2]
        onehot = iota == far
        cx = jnp.sum(jnp.where(onehot, px, 0.0), axis=1, keepdims=True)
        cy = jnp.sum(jnp.where(onehot, py, 0.0), axis=1, keepdims=True)
        cz = jnp.sum(jnp.where(onehot, pz, 0.0), axis=1, keepdims=True)
        sel_out = iota_out == i
        newp_ref[0] = jnp.where(sel_out, cx, newp_ref[0])
        newp_ref[1] = jnp.where(sel_out, cy, newp_ref[1])
        newp_ref[2] = jnp.where(sel_out, cz, newp_ref[2])
        d = (px - cx) ** 2 + (py - cy) ** 2 + (pz - cz) ** 2
        dist = jnp.minimum(dist_ref[...], d)
        dist_ref[...] = dist
        m = jnp.max(dist, axis=1, keepdims=True)
        return jnp.min(jnp.where(dist >= m, iota, N), axis=1, keepdims=True)

    jax.lax.fori_loop(0, NPOINT, body, jnp.zeros((B, 1), jnp.int32))


def _fps_call(xt):
    return pl.pallas_call(
        _fps_kernel,
        in_specs=[pl.BlockSpec((3, B, N), lambda: (0, 0, 0))],
        out_specs=pl.BlockSpec((3, B, NPOINT), lambda: (0, 0, 0)),
        out_shape=jax.ShapeDtypeStruct((3, B, NPOINT), jnp.float32),
        scratch_shapes=[pltpu.VMEM((B, N), jnp.float32)],
    )(xt)


def _query_kernel(q_ref, xc_ref, idx_ref):
    b = pl.program_id(0)
    q = q_ref[0]
    xc = xc_ref[0]
    qq = jnp.sum(q * q, axis=1, keepdims=True)
    xx_c = jnp.sum(xc * xc, axis=0, keepdims=True)
    pd = (2.0 * jnp.dot(q, xc, preferred_element_type=jnp.float32)
          - qq - xx_c)
    iota = jax.lax.broadcasted_iota(jnp.int32, (RQ, N), 1)
    base = b * N
    for t in range(OUT_K):
        m = jnp.max(pd, axis=1, keepdims=True)
        sel = pd >= m
        idxf = jnp.sum(jnp.where(sel, iota, 0), axis=1, keepdims=True)
        idxv = jnp.minimum(idxf, N - 1)
        idx_ref[0, :, pl.ds(t, 1)] = idxv + base
        pd = jnp.where(sel, NEG, pd)


def _query_topk(newp, xt):
    return pl.pallas_call(
        _query_kernel,
        grid=(B, NPOINT // RQ),
        in_specs=[
            pl.BlockSpec((1, RQ, 3), lambda b, r: (b, r, 0)),
            pl.BlockSpec((1, 3, N), lambda b, r: (b, 0, 0)),
        ],
        out_specs=pl.BlockSpec((1, RQ, OUT_K), lambda b, r: (b, r, 0)),
        out_shape=jax.ShapeDtypeStruct((B, NPOINT, OUT_K), jnp.int32),
    )(newp, xt)


def _head_kernel(eb_ref, w_ref, s_ref, b_ref, out_ref):
    acc = jax.lax.dot_general(w_ref[...], eb_ref[0], (((1,), (1,)), ((), ())),
                              preferred_element_type=jnp.float32)
    out_ref[0] = acc * s_ref[...] + b_ref[...]


def _head(ebds, w_all, scale, bias):
    return pl.pallas_call(
        _head_kernel,
        grid=(B,),
        in_specs=[
            pl.BlockSpec((1, NPOINT, 64), lambda b: (b, 0, 0)),
            pl.BlockSpec((384, 64), lambda b: (0, 0)),
            pl.BlockSpec((384, 1), lambda b: (0, 0)),
            pl.BlockSpec((384, 1), lambda b: (0, 0)),
        ],
        out_specs=pl.BlockSpec((1, 384, NPOINT), lambda b: (b, 0, 0)),
        out_shape=jax.ShapeDtypeStruct((B, 384, NPOINT), jnp.float32),
    )(ebds, w_all, scale, bias)


NW = 32
PTS_A = (B * N) // NW
GA = 4
NGRP_A = 128 // GA // 1
CH_A = 128
PTS_B = (B * NPOINT) // NW
GB = 8


def _sc_mesh():
    return plsc.VectorSubcoreMesh(core_axis_name="c", subcore_axis_name="s")


def _max_rows(buf, row0, nrows, c):
    acc = buf[row0, pl.ds(c * 16, 16)]
    for k in range(1, nrows):
        acc = jnp.maximum(acc, buf[row0 + k, pl.ds(c * 16, 16)])
    return acc


def _sum_rows(buf, row0, nrows, c):
    acc = buf[row0, pl.ds(c * 16, 16)]
    for k in range(1, nrows):
        acc = acc + buf[row0 + k, pl.ds(c * 16, 16)]
    return acc


def _scA_body(p_hbm, q_hbm, idx_hbm, out_hbm, idx_v, buf0, buf1, buf2, buf3,
              qbuf, obuf, sem0, sem1, sem2, sem3):
    wid = lax.axis_index("s") * 2 + lax.axis_index("c")
    base = wid * PTS_A
    bufs = (buf0, buf1, buf2, buf3)
    sems = (sem0, sem1, sem2, sem3)

    pltpu.sync_copy(idx_hbm.at[pl.ds(base * K_KNN, PTS_A * K_KNN)],
                    idx_v.at[pl.ds(0, PTS_A * K_KNN)])
    zpad = jnp.zeros((16,), jnp.int32)
    for i in range(3 * GA * K_KNN // 16):
        idx_v[pl.ds(PTS_A * K_KNN + i * 16, 16)] = zpad

    def gdesc(g, r):
        off = g * (GA * K_KNN)
        return pltpu.make_async_copy(
            p_hbm.at[idx_v.at[pl.ds(off, GA * K_KNN)]], bufs[r], sems[r])

    def compute(g, r):
        buf = bufs[r]
        for p in range(GA):
            row = g * GA + p
            for c in range(4):
                acc = _max_rows(buf, p * K_KNN, K_KNN, c)
                acc = acc + qbuf[row, pl.ds(c * 16, 16)]
                obuf[row, pl.ds(c * 16, 16)] = jnp.maximum(acc, 0.2 * acc)

    def chunk_body(ch, carry):
        row0 = base + ch * CH_A
        g0 = ch * NGRP_A
        pltpu.sync_copy(q_hbm.at[pl.ds(row0, CH_A)], qbuf)
        for r in range(3):
            gdesc(g0 + r, r).start()

        def ring_body(j, carry2):
            gj = g0 + 4 * j
            for r in range(4):
                gdesc(gj + r + 3, (r + 3) % 4).start()
                gdesc(gj + r, r).wait()
                compute(4 * j + r, r)
            return carry2

        lax.fori_loop(0, NGRP_A // 4, ring_body, 0)
        for r in range(3):
            gdesc(g0 + NGRP_A + r, r).wait()
        pltpu.sync_copy(obuf, out_hbm.at[pl.ds(row0, CH_A)])
        return carry

    lax.fori_loop(0, PTS_A // CH_A, chunk_body, 0)


def _sc_gather_max(p_flat, q_flat, idx_flat):
    kfn = functools.partial(
        pl.kernel,
        out_type=jax.ShapeDtypeStruct((B * N, 64), jnp.float32),
        mesh=_sc_mesh(),
        compiler_params=pltpu.CompilerParams(use_tc_tiling_on_sc=False),
        scratch_types=[
            pltpu.VMEM((PTS_A * K_KNN + 3 * GA * K_KNN,), jnp.int32),
            pltpu.VMEM((GA * K_KNN, 64), jnp.float32),
            pltpu.VMEM((GA * K_KNN, 64), jnp.float32),
            pltpu.VMEM((GA * K_KNN, 64), jnp.float32),
            pltpu.VMEM((GA * K_KNN, 64), jnp.float32),
            pltpu.VMEM((CH_A, 64), jnp.float32),
            pltpu.VMEM((CH_A, 64), jnp.float32),
            pltpu.SemaphoreType.DMA,
            pltpu.SemaphoreType.DMA,
            pltpu.SemaphoreType.DMA,
            pltpu.SemaphoreType.DMA,
        ],
    )(_scA_body)
    return kfn(p_flat, q_flat, idx_flat)


def _scB_body(x1_hbm, idx_hbm, out_hbm, idx_v, buf0, buf1, buf2, buf3, obuf,
              sem0, sem1, sem2, sem3):
    wid = lax.axis_index("s") * 2 + lax.axis_index("c")
    base = wid * PTS_B
    bufs = (buf0, buf1, buf2, buf3)
    sems = (sem0, sem1, sem2, sem3)

    pltpu.sync_copy(idx_hbm.at[pl.ds(base * OUT_K, PTS_B * OUT_K)],
                    idx_v.at[pl.ds(0, PTS_B * OUT_K)])
    zpad = jnp.zeros((16,), jnp.int32)
    for i in range(3 * GB * OUT_K // 16):
        idx_v[pl.ds(PTS_B * OUT_K + i * 16, 16)] = zpad

    def gdesc(g, r):
        off = g * (GB * OUT_K)
        return pltpu.make_async_copy(
            x1_hbm.at[idx_v.at[pl.ds(off, GB * OUT_K)]], bufs[r], sems[r])

    def compute(g, r):
        buf = bufs[r]
        for p in range(GB):
            row = g * GB + p
            for c in range(4):
                acc = _sum_rows(buf, p * OUT_K, OUT_K, c)
                obuf[row, pl.ds(c * 16, 16)] = acc

    ngrp = PTS_B // GB
    for r in range(3):
        gdesc(r, r).start()

    def ring_body(j, carry2):
        gj = 4 * j
        for r in range(4):
            gdesc(gj + r + 3, (r + 3) % 4).start()
            gdesc(gj + r, r).wait()
            compute(gj + r, r)
        return carry2

    lax.fori_loop(0, ngrp // 4, ring_body, 0)
    for r in range(3):
        gdesc(ngrp + r, r).wait()
    pltpu.sync_copy(obuf, out_hbm.at[pl.ds(base, PTS_B)])


def _sc_gather_sum(x1_flat, idx_flat):
    kfn = functools.partial(
        pl.kernel,
        out_type=jax.ShapeDtypeStruct((B * NPOINT, 64), jnp.float32),
        mesh=_sc_mesh(),
        compiler_params=pltpu.CompilerParams(use_tc_tiling_on_sc=False),
        scratch_types=[
            pltpu.VMEM((PTS_B * OUT_K + 3 * GB * OUT_K,), jnp.int32),
            pltpu.VMEM((GB * OUT_K, 64), jnp.float32),
            pltpu.VMEM((GB * OUT_K, 64), jnp.float32),
            pltpu.VMEM((GB * OUT_K, 64), jnp.float32),
            pltpu.VMEM((GB * OUT_K, 64), jnp.float32),
            pltpu.VMEM((PTS_B, 64), jnp.float32),
            pltpu.SemaphoreType.DMA,
            pltpu.SemaphoreType.DMA,
            pltpu.SemaphoreType.DMA,
            pltpu.SemaphoreType.DMA,
        ],
    )(_scB_body)
    return kfn(x1_flat, idx_flat)


def kernel(x, W1, g1, b1, Wf1, gf1, bf1, Wf2, gf2, bf2, Wf3, gf3, bf3):
    inv = 1.0 / jnp.sqrt(jnp.float32(1.0) + EPS)
    xt = jnp.transpose(x, (0, 2, 1))

    wa = (jnp.transpose(W1[:, :3]) * (g1 * inv)[None, :])
    wq = (jnp.transpose(W1[:, 3:] - W1[:, :3]) * (g1 * inv)[None, :])
    bq = (b1 * jnp.ones((1,), jnp.float32))[None, :]

    idx20, p_rows, q_rows = _knn_topk(x, xt, wa, wq, bq)

    newp3 = _fps_call(jnp.transpose(xt, (1, 0, 2)))
    new_pts = jnp.transpose(newp3, (1, 2, 0))
    idxq = _query_topk(new_pts, xt)

    p_flat = p_rows.reshape(B * N, 64)
    q_flat = q_rows.reshape(B * N, 64)
    x1_flat = _sc_gather_max(p_flat, q_flat, idx20.reshape(-1))
    ebds = _sc_gather_sum(x1_flat, idxq.reshape(-1)).reshape(B, NPOINT, 64)

    w_all = jnp.concatenate([Wf1, Wf2, Wf3], axis=0)
    scale = (jnp.concatenate([gf1, gf2, gf3]) * inv / OUT_K)[:, None]
    bias = jnp.concatenate([bf1, bf2, bf3])[:, None]
    e_all = _head(ebds, w_all, scale, bias)
    e1, e2, e3 = e_all[:, :128], e_all[:, 128:256], e_all[:, 256:]

    mask = jnp.ones((x.shape[0], NPOINT), bool)
    return (e1, e2, e3, new_pts, new_pts, new_pts, mask, mask, mask)


# speedup vs baseline: 1.0937x; 1.0937x over previous
import functools

import jax
import jax.numpy as jnp
from jax import lax
from jax.experimental import pallas as pl
from jax.experimental.pallas import tpu as pltpu
from jax.experimental.pallas import tpu_sc as plsc

B, N, K_KNN, OUT_K, OUT_DIM = 16, 2048, 20, 16, 128
NPOINT = 512
EPS = 1e-5
NEG = -3.0e38

RB = 256
RQ = 256


def _knn_kernel(xr_ref, xc_ref, wa_ref, wq_ref, bq_ref, idx_ref, p_ref, q_ref):
    b = pl.program_id(0)
    xr = xr_ref[0]
    xc = xc_ref[0]
    p_ref[0] = jnp.dot(xr, wa_ref[...], preferred_element_type=jnp.float32)
    q_ref[0] = (jnp.dot(xr, wq_ref[...], preferred_element_type=jnp.float32)
                + bq_ref[...])
    xx_r = jnp.sum(xr * xr, axis=1, keepdims=True)
    xx_c = jnp.sum(xc * xc, axis=0, keepdims=True)
    pd = (2.0 * jnp.dot(xr, xc, preferred_element_type=jnp.float32)
          - xx_r - xx_c)
    iota = jax.lax.broadcasted_iota(jnp.int32, (RB, N), 1)
    base = b * N
    for t in range(K_KNN):
        m = jnp.max(pd, axis=1, keepdims=True)
        sel = pd >= m
        idxf = jnp.sum(jnp.where(sel, iota, 0), axis=1, keepdims=True)
        idxv = jnp.minimum(idxf, N - 1)
        idx_ref[0, :, pl.ds(t, 1)] = idxv + base
        pd = jnp.where(sel, NEG, pd)


def _knn_topk(x, xt, wa, wq, bq):
    return pl.pallas_call(
        _knn_kernel,
        grid=(B, N // RB),
        in_specs=[
            pl.BlockSpec((1, RB, 3), lambda b, r: (b, r, 0)),
            pl.BlockSpec((1, 3, N), lambda b, r: (b, 0, 0)),
            pl.BlockSpec((3, 64), lambda b, r: (0, 0)),
            pl.BlockSpec((3, 64), lambda b, r: (0, 0)),
            pl.BlockSpec((1, 64), lambda b, r: (0, 0)),
        ],
        out_specs=[
            pl.BlockSpec((1, RB, K_KNN), lambda b, r: (b, r, 0)),
            pl.BlockSpec((1, RB, 64), lambda b, r: (b, r, 0)),
            pl.BlockSpec((1, RB, 64), lambda b, r: (b, r, 0)),
        ],
        out_shape=[
            jax.ShapeDtypeStruct((B, N, K_KNN), jnp.int32),
            jax.ShapeDtypeStruct((B, N, 64), jnp.float32),
            jax.ShapeDtypeStruct((B, N, 64), jnp.float32),
        ],
    )(x, xt, wa, wq, bq)


def _fps_kernel(xt_ref, newp_ref, dist_ref):
    dist_ref[...] = jnp.full((B, N), 1e10, jnp.float32)
    iota = jax.lax.broadcasted_iota(jnp.int32, (B, N), 1)
    iota_out = jax.lax.broadcasted_iota(jnp.int32, (B, NPOINT), 1)

    def body(i, far):
        px = xt_ref[0]
        py = xt_ref[1]
        pz = xt_ref[2]
        onehot = iota == far
        cx = jnp.sum(jnp.where(onehot, px, 0.0), axis=1, keepdims=True)
        cy = jnp.sum(jnp.where(onehot, py, 0.0), axis=1, keepdims=True)
        cz = jnp.sum(jnp.where(onehot, pz, 0.0), axis=1, keepdims=True)
        sel_out = iota_out == i
        newp_ref[0] = jnp.where(sel_out, cx, newp_ref[0])
        newp_ref[1] = jnp.where(sel_out, cy, newp_ref[1])
        newp_ref[2] = jnp.where(sel_out, cz, newp_ref[2])
        d = (px - cx) ** 2 + (py - cy) ** 2 + (pz - cz) ** 2
        dist = jnp.minimum(dist_ref[...], d)
        dist_ref[...] = dist
        m = jnp.max(dist, axis=1, keepdims=True)
        return jnp.min(jnp.where(dist >= m, iota, N), axis=1, keepdims=True)

    jax.lax.fori_loop(0, NPOINT, body, jnp.zeros((B, 1), jnp.int32))


def _fps_call(xt):
    return pl.pallas_call(
        _fps_kernel,
        in_specs=[pl.BlockSpec((3, B, N), lambda: (0, 0, 0))],
        out_specs=pl.BlockSpec((3, B, NPOINT), lambda: (0, 0, 0)),
        out_shape=jax.ShapeDtypeStruct((3, B, NPOINT), jnp.float32),
        scratch_shapes=[pltpu.VMEM((B, N), jnp.float32)],
    )(xt)


def _query_kernel(q_ref, xc_ref, idx_ref):
    b = pl.program_id(0)
    q = q_ref[0]
    xc = xc_ref[0]
    qq = jnp.sum(q * q, axis=1, keepdims=True)
    xx_c = jnp.sum(xc * xc, axis=0, keepdims=True)
    pd = (2.0 * jnp.dot(q, xc, preferred_element_type=jnp.float32)
          - qq - xx_c)
    iota = jax.lax.broadcasted_iota(jnp.int32, (RQ, N), 1)
    base = b * N
    for t in range(OUT_K):
        m = jnp.max(pd, axis=1, keepdims=True)
        sel = pd >= m
        idxf = jnp.sum(jnp.where(sel, iota, 0), axis=1, keepdims=True)
        idxv = jnp.minimum(idxf, N - 1)
        idx_ref[0, :, pl.ds(t, 1)] = idxv + base
        pd = jnp.where(sel, NEG, pd)


def _query_topk(newp, xt):
    return pl.pallas_call(
        _query_kernel,
        grid=(B, NPOINT // RQ),
        in_specs=[
            pl.BlockSpec((1, RQ, 3), lambda b, r: (b, r, 0)),
            pl.BlockSpec((1, 3, N), lambda b, r: (b, 0, 0)),
        ],
        out_specs=pl.BlockSpec((1, RQ, OUT_K), lambda b, r: (b, r, 0)),
        out_shape=jax.ShapeDtypeStruct((B, NPOINT, OUT_K), jnp.int32),
    )(newp, xt)


def _head_kernel(eb_ref, w_ref, s_ref, b_ref, out_ref):
    acc = jax.lax.dot_general(w_ref[...], eb_ref[0], (((1,), (1,)), ((), ())),
                              preferred_element_type=jnp.float32)
    out_ref[0] = acc * s_ref[...] + b_ref[...]


def _head(ebds, w_all, scale, bias):
    return pl.pallas_call(
        _head_kernel,
        grid=(B,),
        in_specs=[
            pl.BlockSpec((1, NPOINT, 64), lambda b: (b, 0, 0)),
            pl.BlockSpec((384, 64), lambda b: (0, 0)),
            pl.BlockSpec((384, 1), lambda b: (0, 0)),
            pl.BlockSpec((384, 1), lambda b: (0, 0)),
        ],
        out_specs=pl.BlockSpec((1, 384, NPOINT), lambda b: (b, 0, 0)),
        out_shape=jax.ShapeDtypeStruct((B, 384, NPOINT), jnp.float32),
    )(ebds, w_all, scale, bias)


NW = 32
PTS_A = (B * N) // NW
GA = 4
NGRP_A = 128 // GA // 1
CH_A = 128
PTS_B = (B * NPOINT) // NW
GB = 8


def _sc_mesh():
    return plsc.VectorSubcoreMesh(core_axis_name="c", subcore_axis_name="s")


def _max_rows(buf, row0, nrows, c):
    acc = buf[row0, pl.ds(c * 16, 16)]
    for k in range(1, nrows):
        acc = jnp.maximum(acc, buf[row0 + k, pl.ds(c * 16, 16)])
    return acc


def _sum_rows(buf, row0, nrows, c):
    acc = buf[row0, pl.ds(c * 16, 16)]
    for k in range(1, nrows):
        acc = acc + buf[row0 + k, pl.ds(c * 16, 16)]
    return acc


def _scA_body(p_hbm, q_hbm, idx_hbm, out_hbm, idx_v, buf0, buf1, qbuf, obuf,
              sem0, sem1):
    wid = lax.axis_index("s") * 2 + lax.axis_index("c")
    base = wid * PTS_A

    pltpu.sync_copy(idx_hbm.at[pl.ds(base * K_KNN, PTS_A * K_KNN)],
                    idx_v.at[pl.ds(0, PTS_A * K_KNN)])
    zpad = jnp.zeros((16,), jnp.int32)
    for i in range(GA * K_KNN // 16):
        idx_v[pl.ds(PTS_A * K_KNN + i * 16, 16)] = zpad

    def gdesc(g, buf, sem):
        off = g * (GA * K_KNN)
        return pltpu.make_async_copy(
            p_hbm.at[idx_v.at[pl.ds(off, GA * K_KNN)]], buf, sem)

    def compute(g, buf):
        for p in range(GA):
            row = g * GA + p
            for c in range(4):
                acc = _max_rows(buf, p * K_KNN, K_KNN, c)
                acc = acc + qbuf[row, pl.ds(c * 16, 16)]
                obuf[row, pl.ds(c * 16, 16)] = jnp.maximum(acc, 0.2 * acc)

    def chunk_body(ch, carry):
        row0 = base + ch * CH_A
        pltpu.sync_copy(q_hbm.at[pl.ds(row0, CH_A)], qbuf)
        gdesc(ch * NGRP_A, buf0, sem0).start()

        def pair_body(j, carry2):
            g0 = ch * NGRP_A + 2 * j
            gdesc(g0 + 1, buf1, sem1).start()
            gdesc(g0, buf0, sem0).wait()
            compute(2 * j, buf0)
            gdesc(g0 + 2, buf0, sem0).start()
            gdesc(g0 + 1, buf1, sem1).wait()
            compute(2 * j + 1, buf1)
            return carry2

        lax.fori_loop(0, NGRP_A // 2, pair_body, 0)
        gdesc((ch + 1) * NGRP_A, buf0, sem0).wait()
        pltpu.sync_copy(obuf, out_hbm.at[pl.ds(row0, CH_A)])
        return carry

    lax.fori_loop(0, PTS_A // CH_A, chunk_body, 0)


def _sc_gather_max(p_flat, q_flat, idx_flat):
    kfn = functools.partial(
        pl.kernel,
        out_type=jax.ShapeDtypeStruct((B * N, 64), jnp.float32),
        mesh=_sc_mesh(),
        compiler_params=pltpu.CompilerParams(use_tc_tiling_on_sc=False),
        scratch_types=[
            pltpu.VMEM((PTS_A * K_KNN + GA * K_KNN,), jnp.int32),
            pltpu.VMEM((GA * K_KNN, 64), jnp.float32),
            pltpu.VMEM((GA * K_KNN, 64), jnp.float32),
            pltpu.VMEM((CH_A, 64), jnp.float32),
            pltpu.VMEM((CH_A, 64), jnp.float32),
            pltpu.SemaphoreType.DMA,
            pltpu.SemaphoreType.DMA,
        ],
    )(_scA_body)
    return kfn(p_flat, q_flat, idx_flat)


def _scB_body(x1_hbm, idx_hbm, out_hbm, idx_v, buf0, buf1, obuf, sem0, sem1):
    wid = lax.axis_index("s") * 2 + lax.axis_index("c")
    base = wid * PTS_B

    pltpu.sync_copy(idx_hbm.at[pl.ds(base * OUT_K, PTS_B * OUT_K)],
                    idx_v.at[pl.ds(0, PTS_B * OUT_K)])
    zpad = jnp.zeros((16,), jnp.int32)
    for i in range(GB * OUT_K // 16):
        idx_v[pl.ds(PTS_B * OUT_K + i * 16, 16)] = zpad

    def gdesc(g, buf, sem):
        off = g * (GB * OUT_K)
        return pltpu.make_async_copy(
            x1_hbm.at[idx_v.at[pl.ds(off, GB * OUT_K)]], buf, sem)

    def compute(g, buf):
        for p in range(GB):
            row = g * GB + p
            for c in range(4):
                acc = _sum_rows(buf, p * OUT_K, OUT_K, c)
                obuf[row, pl.ds(c * 16, 16)] = acc

    ngrp = PTS_B // GB
    gdesc(0, buf0, sem0).start()

    def pair_body(j, carry2):
        g0 = 2 * j
        gdesc(g0 + 1, buf1, sem1).start()
        gdesc(g0, buf0, sem0).wait()
        compute(g0, buf0)
        gdesc(g0 + 2, buf0, sem0).start()
        gdesc(g0 + 1, buf1, sem1).wait()
        compute(g0 + 1, buf1)
        return carry2

    lax.fori_loop(0, ngrp // 2, pair_body, 0)
    gdesc(ngrp, buf0, sem0).wait()
    pltpu.sync_copy(obuf, out_hbm.at[pl.ds(base, PTS_B)])


def _sc_gather_sum(x1_flat, idx_flat):
    kfn = functools.partial(
        pl.kernel,
        out_type=jax.ShapeDtypeStruct((B * NPOINT, 64), jnp.float32),
        mesh=_sc_mesh(),
        compiler_params=pltpu.CompilerParams(use_tc_tiling_on_sc=False),
        scratch_types=[
            pltpu.VMEM((PTS_B * OUT_K + GB * OUT_K,), jnp.int32),
            pltpu.VMEM((GB * OUT_K, 64), jnp.float32),
            pltpu.VMEM((GB * OUT_K, 64), jnp.float32),
            pltpu.VMEM((PTS_B, 64), jnp.float32),
            pltpu.SemaphoreType.DMA,
            pltpu.SemaphoreType.DMA,
        ],
    )(_scB_body)
    return kfn(x1_flat, idx_flat)


def kernel(x, W1, g1, b1, Wf1, gf1, bf1, Wf2, gf2, bf2, Wf3, gf3, bf3):
    inv = 1.0 / jnp.sqrt(jnp.float32(1.0) + EPS)
    xt = jnp.transpose(x, (0, 2, 1))

    wa = (jnp.transpose(W1[:, :3]) * (g1 * inv)[None, :])
    wq = (jnp.transpose(W1[:, 3:] - W1[:, :3]) * (g1 * inv)[None, :])
    bq = (b1 * jnp.ones((1,), jnp.float32))[None, :]

    idx20, p_rows, q_rows = _knn_topk(x, xt, wa, wq, bq)

    newp3 = _fps_call(jnp.transpose(xt, (1, 0, 2)))
    new_pts = jnp.transpose(newp3, (1, 2, 0))
    idxq = _query_topk(new_pts, xt)

    p_flat = p_rows.reshape(B * N, 64)
    q_flat = q_rows.reshape(B * N, 64)
    x1_flat = _sc_gather_max(p_flat, q_flat, idx20.reshape(-1))
    ebds = _sc_gather_sum(x1_flat, idxq.reshape(-1)).reshape(B, NPOINT, 64)

    w_all = jnp.concatenate([Wf1, Wf2, Wf3], axis=0)
    scale = (jnp.concatenate([gf1, gf2, gf3]) * inv / OUT_K)[:, None]
    bias = jnp.concatenate([bf1, bf2, bf3])[:, None]
    e_all = _head(ebds, w_all, scale, bias)
    e1, e2, e3 = e_all[:, :128], e_all[:, 128:256], e_all[:, 256:]

    mask = jnp.ones((x.shape[0], NPOINT), bool)
    return (e1, e2, e3, new_pts, new_pts, new_pts, mask, mask, mask)
